# Initial kernel scaffold; baseline (speedup 1.0000x reference)
import jax
import jax.numpy as jnp
from jax.experimental import pallas as pl


def kernel(u, v, adj, rel, usr_table, ent_table, rel_table, W0, b0, W1, b1):
    raise NotImplementedError("write your pallas kernel here")


# speedup vs baseline: 3.1892x; 3.1892x over previous
import jax
import jax.numpy as jnp
from jax import lax
from jax.experimental import pallas as pl
from jax.experimental.pallas import tpu as pltpu
from jax.experimental.pallas import tpu_sc as plsc

B = 1024
K = 16
D = 128
NW = 32
BPW = B // NW
SPW = BPW * K
L = 16


def _sc_body(u_h, v_h, adj_h, usr_h, ent_h,
             U_h, E0_h, E1_h, S2_h,
             vbuf, ubuf, vdiv8, vpad, adjv, e1idx, e1div8, e2big, e2idx,
             entrows, e1rows, sumrows, urows, sem):
    cid = lax.axis_index("c")
    sid = lax.axis_index("s")
    wid = sid * 2 + cid
    base = wid * BPW
    sbase = wid * SPW

    pltpu.sync_copy(v_h.at[pl.ds(base, BPW)], vbuf)
    pltpu.sync_copy(u_h.at[pl.ds(base, BPW)], ubuf)

    pltpu.async_copy(usr_h.at[ubuf], urows, sem).wait()
    pltpu.sync_copy(urows, U_h.at[pl.ds(base, BPW)])
    pltpu.async_copy(ent_h.at[vbuf], urows, sem).wait()
    pltpu.sync_copy(urows, E0_h.at[pl.ds(base, BPW)])

    for g in range(BPW // L):
        vv = vbuf[pl.ds(g * L, L)]
        vdiv8[pl.ds(g * L, L)] = vv >> 3
        vpad[pl.ds(g * L, L)] = vv
    pltpu.async_copy(adj_h.at[vdiv8], adjv, sem).wait()

    @pl.loop(0, BPW, unroll=8)
    def _extract1(r):
        off = (vpad[pl.ds(r, L)][0] & 7) * K
        e1idx[pl.ds(r * K, K)] = adjv[r, pl.ds(off, K)]

    for g in range(SPW // L):
        e1div8[pl.ds(g * L, L)] = e1idx[pl.ds(g * L, L)] >> 3

    for c in range(4):
        pltpu.async_copy(adj_h.at[e1div8.at[pl.ds(c * 128, 128)]],
                         e2big, sem).wait()

        @pl.loop(0, 128, unroll=8)
        def _extract2(r, c=c):
            p = c * 128 + r
            off = (e1idx[pl.ds(p, L)][0] & 7) * K
            e2idx[pl.ds(p * K, K)] = e2big[r, pl.ds(off, K)]

    for c in range(4):
        pltpu.async_copy(ent_h.at[e1idx.at[pl.ds(c * 128, 128)]],
                         e1rows, sem).wait()
        pltpu.sync_copy(e1rows, E1_h.at[pl.ds(sbase + c * 128, 128)])

    @pl.loop(0, 64)
    def _chunk(t):
        pltpu.async_copy(ent_h.at[e2idx.at[pl.ds(t * 128, 128)]],
                         entrows, sem).wait()
        for s in range(8):
            for d in range(8):
                acc = entrows[s * 16, pl.ds(d * L, L)]
                for k in range(1, 16):
                    acc = acc + entrows[s * 16 + k, pl.ds(d * L, L)]
                sumrows[s, pl.ds(d * L, L)] = acc
        pltpu.sync_copy(sumrows, S2_h.at[pl.ds(sbase + t * 8, 8)])


def _sc_gather(u, v, adj128, usr_table, ent_table):
    mesh = plsc.VectorSubcoreMesh(core_axis_name="c", subcore_axis_name="s")
    f32 = jnp.float32
    kern = pl.kernel(
        _sc_body,
        out_type=(
            jax.ShapeDtypeStruct((B, D), f32),
            jax.ShapeDtypeStruct((B, D), f32),
            jax.ShapeDtypeStruct((B * K, D), f32),
            jax.ShapeDtypeStruct((B * K, D), f32),
        ),
        mesh=mesh,
        scratch_types=[
            pltpu.VMEM((BPW,), jnp.int32),
            pltpu.VMEM((BPW,), jnp.int32),
            pltpu.VMEM((BPW,), jnp.int32),
            pltpu.VMEM((BPW + L,), jnp.int32),
            pltpu.VMEM((BPW, 128), jnp.int32),
            pltpu.VMEM((SPW + L,), jnp.int32),
            pltpu.VMEM((SPW,), jnp.int32),
            pltpu.VMEM((128, 128), jnp.int32),
            pltpu.VMEM((SPW * K,), jnp.int32),
            pltpu.VMEM((128, D), f32),
            pltpu.VMEM((128, D), f32),
            pltpu.VMEM((8, D), f32),
            pltpu.VMEM((BPW, D), f32),
            pltpu.SemaphoreType.DMA,
        ],
    )
    return kern(u, v, adj128, usr_table, ent_table)


def _tc_body(u_ref, e0_ref, e1_ref, s2_ref, w0_ref, b0_ref, w1_ref, b1_ref,
             out_ref):
    f32 = jnp.float32
    bb = e0_ref.shape[0]
    w0 = w0_ref[...]
    b0 = b0_ref[...]
    comb1 = e1_ref[...] + s2_ref[...] * (1.0 / K)
    x1 = jnp.maximum(jnp.dot(comb1, w0, preferred_element_type=f32) + b0, 0.0)
    m0 = jnp.mean(e1_ref[...].reshape(bb, K, D), axis=1)
    x0 = jnp.maximum(
        jnp.dot(e0_ref[...] + m0, w0, preferred_element_type=f32) + b0, 0.0)
    m1 = jnp.mean(x1.reshape(bb, K, D), axis=1)
    item = jnp.tanh(
        jnp.dot(x0 + m1, w1_ref[...], preferred_element_type=f32) + b1_ref[...])
    out_ref[...] = jnp.sum(u_ref[...] * item, axis=1)


def _tc_dense(U, E0, E1, S2, W0, b0, W1, b1):
    BB = 128
    grid = B // BB
    return pl.pallas_call(
        _tc_body,
        grid=(grid,),
        in_specs=[
            pl.BlockSpec((BB, D), lambda i: (i, 0)),
            pl.BlockSpec((BB, D), lambda i: (i, 0)),
            pl.BlockSpec((BB * K, D), lambda i: (i, 0)),
            pl.BlockSpec((BB * K, D), lambda i: (i, 0)),
            pl.BlockSpec((D, D), lambda i: (0, 0)),
            pl.BlockSpec((1, D), lambda i: (0, 0)),
            pl.BlockSpec((D, D), lambda i: (0, 0)),
            pl.BlockSpec((1, D), lambda i: (0, 0)),
        ],
        out_specs=pl.BlockSpec((BB,), lambda i: (i,)),
        out_shape=jax.ShapeDtypeStruct((B,), jnp.float32),
    )(U, E0, E1, S2, W0, b0, W1, b1)


def kernel(u, v, adj, rel, usr_table, ent_table, rel_table, W0, b0, W1, b1):
    del rel, rel_table
    u = u.astype(jnp.int32)
    v = v.astype(jnp.int32)
    adj128 = adj.astype(jnp.int32).reshape(-1, 128)
    U, E0, E1, S2 = _sc_gather(u, v, adj128, usr_table, ent_table)
    return _tc_dense(U, E0, E1, S2, W0, b0.reshape(1, D), W1, b1.reshape(1, D))
